# baseline (speedup 1.0000x reference)
import functools

import jax
import jax.numpy as jnp
from jax.experimental import pallas as pl
from jax.experimental.pallas import tpu as pltpu

H = 64
BE = 2560


def _silu(v):
    return v * jax.nn.sigmoid(v)


def _init_body(x_ref, w_ref, b_ref, w1a_ref, w1b_ref, h_ref, arow_ref, acol_ref):
    h = jnp.dot(x_ref[...], w_ref[...], preferred_element_type=jnp.float32) + b_ref[...]
    h_ref[...] = h
    arow_ref[...] = jnp.dot(h, w1a_ref[...], preferred_element_type=jnp.float32)
    acol_ref[...] = jnp.dot(h, w1b_ref[...], preferred_element_type=jnp.float32)


def _init_call(x, w, b, w1a, w1b):
    n = x.shape[0]
    return pl.pallas_call(
        _init_body,
        out_shape=[
            jax.ShapeDtypeStruct((n, H), jnp.float32),
            jax.ShapeDtypeStruct((n, H), jnp.float32),
            jax.ShapeDtypeStruct((n, H), jnp.float32),
        ],
    )(x, w, b, w1a, w1b)


def _edge_body(esum_ref, cd_ref, ea_ref, wr_ref, wa_ref, b1_ref, w2_ref,
               b2_ref, wc1_ref, bc1_ref, wc2_ref, m_ref, trans_ref):
    cd = cd_ref[...]
    rad = jnp.sum(cd * cd, axis=1, keepdims=True)
    einp = (esum_ref[...] + rad * wr_ref[...] + ea_ref[...] * wa_ref[...]
            + b1_ref[...])
    m1 = _silu(einp)
    m = _silu(jnp.dot(m1, w2_ref[...], preferred_element_type=jnp.float32)
              + b2_ref[...])
    c = _silu(jnp.dot(m, wc1_ref[...], preferred_element_type=jnp.float32)
              + bc1_ref[...])
    cw = jnp.dot(c, wc2_ref[...], preferred_element_type=jnp.float32)
    m_ref[...] = m
    trans_ref[...] = cd * cw


def _edge_call(esum, cd4, ea, lp):
    e = esum.shape[0]
    grid = (e // BE,)
    blk = lambda i: (i, 0)
    rep = lambda i: (0, 0)
    return pl.pallas_call(
        _edge_body,
        grid=grid,
        in_specs=[
            pl.BlockSpec((BE, H), blk),
            pl.BlockSpec((BE, 4), blk),
            pl.BlockSpec((BE, 1), blk),
            pl.BlockSpec((1, H), rep),
            pl.BlockSpec((1, H), rep),
            pl.BlockSpec((1, H), rep),
            pl.BlockSpec((H, H), rep),
            pl.BlockSpec((1, H), rep),
            pl.BlockSpec((H, H), rep),
            pl.BlockSpec((1, H), rep),
            pl.BlockSpec((H, 1), rep),
        ],
        out_specs=[
            pl.BlockSpec((BE, H), blk),
            pl.BlockSpec((BE, 4), blk),
        ],
        out_shape=[
            jax.ShapeDtypeStruct((e, H), jnp.float32),
            jax.ShapeDtypeStruct((e, 4), jnp.float32),
        ],
    )(esum, cd4, ea, lp["wr"], lp["wa"], lp["b1"], lp["w2"], lp["b2"],
      lp["wc1"], lp["bc1"], lp["wc2"])


def _node_body(h_ref, agg_ref, ts_ref, cnt_ref, coord_ref, wn1a_ref, wn1b_ref,
               bn1_ref, wn2_ref, bn2_ref, w1a_ref, w1b_ref,
               h_ref_o, coord_ref_o, arow_ref, acol_ref):
    h = h_ref[...]
    u1 = _silu(jnp.dot(h, wn1a_ref[...], preferred_element_type=jnp.float32)
               + jnp.dot(agg_ref[...], wn1b_ref[...],
                         preferred_element_type=jnp.float32)
               + bn1_ref[...])
    upd = jnp.dot(u1, wn2_ref[...], preferred_element_type=jnp.float32) + bn2_ref[...]
    hn = h + upd
    h_ref_o[...] = hn
    coord_ref_o[...] = coord_ref[...] + ts_ref[...] / jnp.clip(cnt_ref[...], 1.0)
    arow_ref[...] = jnp.dot(hn, w1a_ref[...], preferred_element_type=jnp.float32)
    acol_ref[...] = jnp.dot(hn, w1b_ref[...], preferred_element_type=jnp.float32)


def _node_call(h, agg, ts4, cnt, coord4, lp, w1a_next, w1b_next):
    n = h.shape[0]
    return pl.pallas_call(
        _node_body,
        out_shape=[
            jax.ShapeDtypeStruct((n, H), jnp.float32),
            jax.ShapeDtypeStruct((n, 4), jnp.float32),
            jax.ShapeDtypeStruct((n, H), jnp.float32),
            jax.ShapeDtypeStruct((n, H), jnp.float32),
        ],
    )(h, agg, ts4, cnt, coord4, lp["wn1a"], lp["wn1b"], lp["bn1"], lp["wn2"],
      lp["bn2"], w1a_next, w1b_next)


def _tail_body(h_ref, agg_ref, wn1a_ref, wn1b_ref, bn1_ref, wn2_ref, bn2_ref,
               wo_ref, bo_ref, wf_ref, bf_ref, out_ref):
    h = h_ref[...]
    u1 = _silu(jnp.dot(h, wn1a_ref[...], preferred_element_type=jnp.float32)
               + jnp.dot(agg_ref[...], wn1b_ref[...],
                         preferred_element_type=jnp.float32)
               + bn1_ref[...])
    hn = h + jnp.dot(u1, wn2_ref[...], preferred_element_type=jnp.float32) + bn2_ref[...]
    ho = jnp.dot(hn, wo_ref[...], preferred_element_type=jnp.float32) + bo_ref[...]
    g = jnp.mean(ho, axis=0, keepdims=True)
    out_ref[...] = jnp.dot(g, wf_ref[...], preferred_element_type=jnp.float32) + bf_ref[...]


def _tail_call(h, agg, lp, wo, bo, wf, bf):
    d_out = wf.shape[1]
    return pl.pallas_call(
        _tail_body,
        out_shape=jax.ShapeDtypeStruct((1, d_out), jnp.float32),
    )(h, agg, lp["wn1a"], lp["wn1b"], lp["bn1"], lp["wn2"], lp["bn2"],
      wo, bo, wf, bf)


def _prep_layer(lp):
    w1 = lp["e1"]["W"]
    n1 = lp["n1"]["W"]
    return {
        "w1a": w1[:H],
        "w1b": w1[H:2 * H],
        "wr": w1[2 * H:2 * H + 1],
        "wa": w1[2 * H + 1:2 * H + 2],
        "b1": lp["e1"]["b"][None, :],
        "w2": lp["e2"]["W"],
        "b2": lp["e2"]["b"][None, :],
        "wc1": lp["c1"]["W"],
        "bc1": lp["c1"]["b"][None, :],
        "wc2": lp["c2"]["W"],
        "wn1a": n1[:H],
        "wn1b": n1[H:],
        "bn1": lp["n1"]["b"][None, :],
        "wn2": lp["n2"]["W"],
        "bn2": lp["n2"]["b"][None, :],
    }


def kernel(x, edge_index, coord, edge_attr, params):
    row, col = edge_index[0], edge_index[1]
    n = x.shape[0]
    lps = [_prep_layer(lp) for lp in params["layers"]]

    h, arow, acol = _init_call(
        x, params["emb_in"]["W"], params["emb_in"]["b"][None, :],
        lps[0]["w1a"], lps[0]["w1b"])

    coord4 = jnp.pad(coord, ((0, 0), (0, 1)))
    cnt = jax.ops.segment_sum(jnp.ones((row.shape[0], 1), jnp.float32), row,
                              num_segments=n)

    for li, lp in enumerate(lps):
        esum = arow[row] + acol[col]
        cd4 = coord4[row] - coord4[col]
        m, trans = _edge_call(esum, cd4, edge_attr, lp)
        agg = jax.ops.segment_sum(m, row, num_segments=n)
        if li + 1 < len(lps):
            ts4 = jax.ops.segment_sum(trans, row, num_segments=n)
            h, coord4, arow, acol = _node_call(
                h, agg, ts4, cnt, coord4, lp,
                lps[li + 1]["w1a"], lps[li + 1]["w1b"])
        else:
            out = _tail_call(h, agg, lp, params["emb_out"]["W"],
                             params["emb_out"]["b"][None, :],
                             params["fc"]["W"], params["fc"]["b"][None, :])
    return out


# speedup vs baseline: 3.1389x; 3.1389x over previous
import functools

import jax
import jax.numpy as jnp
from jax import lax
from jax.experimental import pallas as pl
from jax.experimental.pallas import tpu as pltpu
from jax.experimental.pallas import tpu_sc as plsc

H = 64
W = 128
N_NODES = 10000
N_PAD = 10112
E_EDGES = 320000
E_PAD = 327680
NC, NS, LANES = 2, 16, 16
NW = NC * NS
EPT = E_PAD // NW
CH = 256
SCH = 256
TCH = 128
NPAIR = EPT // 1024
RPS = N_PAD // NS
BE = 2560

_MESH = plsc.VectorSubcoreMesh(core_axis_name="c", subcore_axis_name="s",
                               num_cores=NC, num_subcores=NS)


def _silu(v):
    return v * jax.nn.sigmoid(v)


def _sc_gather_body(ta_h, tb_h, rowg_h, colg_h, ga_h, gb_h,
                    ridx, cidx, bufa, bufb, sema, semb):
    wid = lax.axis_index("s") * NC + lax.axis_index("c")
    base = wid * EPT
    gbase0 = wid * (EPT // 128)

    def group(gi, carry):
        pltpu.sync_copy(rowg_h.at[pl.ds(gbase0 + gi * 8, 8)], ridx)
        pltpu.sync_copy(colg_h.at[pl.ds(gbase0 + gi * 8, 8)], cidx)
        for q in range(1024 // CH):
            ebase = base + gi * 1024 + q * CH
            cps = []
            for j in range(CH // 128):
                jr = q * (CH // 128) + j
                d = pl.ds(j * 128, 128)
                cps.append(pltpu.async_copy(ta_h.at[ridx.at[jr]], bufa.at[d], sema))
                cps.append(pltpu.async_copy(tb_h.at[cidx.at[jr]], bufb.at[d], semb))
            for cp_ in cps:
                cp_.wait()
            es = pl.ds(ebase, CH)
            pltpu.sync_copy(bufa, ga_h.at[es])
            pltpu.sync_copy(bufb, gb_h.at[es])
        return carry

    lax.fori_loop(0, NPAIR, group, 0)


def _sc_gather(ta, tb, rowg, colg):
    f = pl.kernel(
        _sc_gather_body,
        out_type=[
            jax.ShapeDtypeStruct((E_PAD, W), jnp.float32),
            jax.ShapeDtypeStruct((E_PAD, W), jnp.float32),
        ],
        mesh=_MESH,
        scratch_types=[
            pltpu.VMEM((8, 128), jnp.int32),
            pltpu.VMEM((8, 128), jnp.int32),
            pltpu.VMEM((CH, W), jnp.float32),
            pltpu.VMEM((CH, W), jnp.float32),
            pltpu.SemaphoreType.DMA,
            pltpu.SemaphoreType.DMA,
        ],
    )
    return f(ta, tb, rowg, colg)


def _sc_scatter_body(mt_h, rows_h, aggp_h, idx, mbuf, aggs):
    cid = lax.axis_index("c")
    sid = lax.axis_index("s")
    wid = sid * NC + cid
    base = wid * EPT

    def zrow(r, carry):
        for c in range(W // LANES):
            mbuf[r, pl.ds(c * LANES, LANES)] = jnp.zeros((LANES,), jnp.float32)
        return carry

    lax.fori_loop(0, SCH, zrow, 0)
    ob = sid * RPS
    full = RPS // SCH
    for i in range(full):
        pltpu.sync_copy(mbuf, aggs.at[pl.ds(ob + i * SCH, SCH)])
    rem = RPS - full * SCH
    if rem:
        pltpu.sync_copy(mbuf.at[pl.ds(0, rem)],
                        aggs.at[pl.ds(ob + full * SCH, rem)])
    plsc.subcore_barrier()
    gbase0 = wid * (EPT // 128)

    def group(gi, carry):
        pltpu.sync_copy(rows_h.at[pl.ds(gbase0 + gi * 8, 8)], idx)
        for q in range(1024 // SCH):
            ebase = base + gi * 1024 + q * SCH
            pltpu.sync_copy(mt_h.at[pl.ds(ebase, SCH)], mbuf)
            for j in range(SCH // 128):
                jr = q * (SCH // 128) + j
                d = pl.ds(j * 128, 128)
                pltpu.sync_copy(mbuf.at[d], aggs.at[idx.at[jr]], add=True)
        return carry

    lax.fori_loop(0, NPAIR, group, 0)
    plsc.subcore_barrier()
    rs = pl.ds(sid * RPS, RPS)
    pltpu.sync_copy(aggs.at[rs], aggp_h.at[cid].at[rs])


def _sc_scatter(mt, rows):
    f = pl.kernel(
        _sc_scatter_body,
        out_type=jax.ShapeDtypeStruct((NC, N_PAD, W), jnp.float32),
        mesh=_MESH,
        scratch_types=[
            pltpu.VMEM((8, 128), jnp.int32),
            pltpu.VMEM((SCH, W), jnp.float32),
            pltpu.VMEM_SHARED((N_PAD, W), jnp.float32),
        ],
    )
    return f(mt, rows)


def _init_body(x_ref, c16_ref, w_ref, b_ref, w1a_ref, w1b_ref,
               h_ref, ta_ref, tb_ref):
    h = jnp.dot(x_ref[...], w_ref[...], preferred_element_type=jnp.float32) + b_ref[...]
    h_ref[...] = h
    n = h.shape[0]
    z = jnp.zeros((n, W - H - LANES), jnp.float32)
    c16 = c16_ref[...]
    ta = jnp.dot(h, w1a_ref[...], preferred_element_type=jnp.float32)
    tb = jnp.dot(h, w1b_ref[...], preferred_element_type=jnp.float32)
    ta_ref[...] = jnp.concatenate([ta, c16, z], axis=1)
    tb_ref[...] = jnp.concatenate([tb, c16, z], axis=1)


def _init_call(x, coord16, w, b, w1a, w1b):
    n = x.shape[0]
    return pl.pallas_call(
        _init_body,
        out_shape=[
            jax.ShapeDtypeStruct((n, H), jnp.float32),
            jax.ShapeDtypeStruct((n, W), jnp.float32),
            jax.ShapeDtypeStruct((n, W), jnp.float32),
        ],
    )(x, coord16, w, b, w1a, w1b)


def _edge_body(ga_ref, gb_ref, ea_ref, wr_ref, wa_ref,
               b1_ref, w2_ref, b2_ref, wc1_ref, bc1_ref, wc2_ref, mt_ref):
    ga = ga_ref[...]
    gb = gb_ref[...]
    cd = ga[:, H:H + LANES] - gb[:, H:H + LANES]
    rad = jnp.sum(cd * cd, axis=1, keepdims=True)
    einp = (ga[:, :H] + gb[:, :H] + rad * wr_ref[...]
            + ea_ref[...] * wa_ref[...] + b1_ref[...])
    m1 = _silu(einp)
    m = _silu(jnp.dot(m1, w2_ref[...], preferred_element_type=jnp.float32)
              + b2_ref[...])
    c = _silu(jnp.dot(m, wc1_ref[...], preferred_element_type=jnp.float32)
              + bc1_ref[...])
    cw = jnp.dot(c, wc2_ref[...], preferred_element_type=jnp.float32)
    lane = lax.broadcasted_iota(jnp.int32, (BE, LANES), 1)
    trans = jnp.where(lane == 3, 1.0, cd * cw)
    z = jnp.zeros((BE, W - H - LANES), jnp.float32)
    mt_ref[...] = jnp.concatenate([m, trans, z], axis=1)


def _edge_call(ga, gb, ea, lp):
    e = ga.shape[0]
    grid = (e // BE,)
    blk = lambda i: (i, 0)
    rep = lambda i: (0, 0)
    return pl.pallas_call(
        _edge_body,
        grid=grid,
        in_specs=[
            pl.BlockSpec((BE, W), blk),
            pl.BlockSpec((BE, W), blk),
            pl.BlockSpec((BE, 1), blk),
            pl.BlockSpec((1, H), rep),
            pl.BlockSpec((1, H), rep),
            pl.BlockSpec((1, H), rep),
            pl.BlockSpec((H, H), rep),
            pl.BlockSpec((1, H), rep),
            pl.BlockSpec((H, H), rep),
            pl.BlockSpec((1, H), rep),
            pl.BlockSpec((H, 1), rep),
        ],
        out_specs=pl.BlockSpec((BE, W), blk),
        out_shape=jax.ShapeDtypeStruct((e, W), jnp.float32),
    )(ga, gb, ea, lp["wr"], lp["wa"], lp["b1"], lp["w2"], lp["b2"],
      lp["wc1"], lp["bc1"], lp["wc2"])


def _node_body(h_ref, tap_ref, p0_ref, p1_ref,
               wn1a_ref, wn1b_ref, bn1_ref,
               wn2_ref, bn2_ref, w1a_ref, w1b_ref,
               h_ref_o, ta_ref, tb_ref):
    h = h_ref[...]
    n = h.shape[0]
    p = (p0_ref[...] + p1_ref[...])[:n]
    agg = p[:, :H]
    ts = p[:, H:H + LANES]
    cnt = jnp.clip(ts[:, 3:4], 1.0)
    lane = lax.broadcasted_iota(jnp.int32, ts.shape, 1)
    c16 = tap_ref[...][:, H:H + LANES] + jnp.where(lane < 3, ts / cnt, 0.0)
    u1 = _silu(jnp.dot(h, wn1a_ref[...], preferred_element_type=jnp.float32)
               + jnp.dot(agg, wn1b_ref[...], preferred_element_type=jnp.float32)
               + bn1_ref[...])
    hn = h + jnp.dot(u1, wn2_ref[...], preferred_element_type=jnp.float32) + bn2_ref[...]
    h_ref_o[...] = hn
    z = jnp.zeros((n, W - H - LANES), jnp.float32)
    ta = jnp.dot(hn, w1a_ref[...], preferred_element_type=jnp.float32)
    tb = jnp.dot(hn, w1b_ref[...], preferred_element_type=jnp.float32)
    ta_ref[...] = jnp.concatenate([ta, c16, z], axis=1)
    tb_ref[...] = jnp.concatenate([tb, c16, z], axis=1)


def _node_call(h, ta_prev, p0, p1, lp, w1a_next, w1b_next):
    n = h.shape[0]
    return pl.pallas_call(
        _node_body,
        out_shape=[
            jax.ShapeDtypeStruct((n, H), jnp.float32),
            jax.ShapeDtypeStruct((n, W), jnp.float32),
            jax.ShapeDtypeStruct((n, W), jnp.float32),
        ],
    )(h, ta_prev, p0, p1, lp["wn1a"], lp["wn1b"], lp["bn1"],
      lp["wn2"], lp["bn2"], w1a_next, w1b_next)


def _tail_body(h_ref, p0_ref, p1_ref, wn1a_ref, wn1b_ref, bn1_ref,
               wn2_ref, bn2_ref, wo_ref, bo_ref, wf_ref, bf_ref, out_ref):
    h = h_ref[...]
    n = h.shape[0]
    agg = (p0_ref[...] + p1_ref[...])[:n, :H]
    u1 = _silu(jnp.dot(h, wn1a_ref[...], preferred_element_type=jnp.float32)
               + jnp.dot(agg, wn1b_ref[...], preferred_element_type=jnp.float32)
               + bn1_ref[...])
    hn = h + jnp.dot(u1, wn2_ref[...], preferred_element_type=jnp.float32) + bn2_ref[...]
    ho = jnp.dot(hn, wo_ref[...], preferred_element_type=jnp.float32) + bo_ref[...]
    g = jnp.mean(ho, axis=0, keepdims=True)
    out_ref[...] = jnp.dot(g, wf_ref[...], preferred_element_type=jnp.float32) + bf_ref[...]


def _tail_call(h, p0, p1, lp, wo, bo, wf, bf):
    d_out = wf.shape[1]
    return pl.pallas_call(
        _tail_body,
        out_shape=jax.ShapeDtypeStruct((1, d_out), jnp.float32),
    )(h, p0, p1, lp["wn1a"], lp["wn1b"], lp["bn1"], lp["wn2"], lp["bn2"],
      wo, bo, wf, bf)


def _prep_layer(lp):
    w1 = lp["e1"]["W"]
    n1 = lp["n1"]["W"]
    return {
        "w1a": w1[:H],
        "w1b": w1[H:2 * H],
        "wr": w1[2 * H:2 * H + 1],
        "wa": w1[2 * H + 1:2 * H + 2],
        "b1": lp["e1"]["b"][None, :],
        "w2": lp["e2"]["W"],
        "b2": lp["e2"]["b"][None, :],
        "wc1": lp["c1"]["W"],
        "bc1": lp["c1"]["b"][None, :],
        "wc2": lp["c2"]["W"],
        "wn1a": n1[:H],
        "wn1b": n1[H:],
        "bn1": lp["n1"]["b"][None, :],
        "wn2": lp["n2"]["W"],
        "bn2": lp["n2"]["b"][None, :],
    }


def kernel(x, edge_index, coord, edge_attr, params):
    row, col = edge_index[0], edge_index[1]
    pe = E_PAD - E_EDGES
    rowg = jnp.pad(row, (0, pe)).reshape(E_PAD // 128, 128)
    colg = jnp.pad(col, (0, pe)).reshape(E_PAD // 128, 128)
    rows = jnp.pad(row, (0, pe), constant_values=N_NODES).reshape(E_PAD // 128, 128)
    ea_pad = jnp.pad(edge_attr, ((0, pe), (0, 0)))
    coord16 = jnp.pad(coord, ((0, 0), (0, LANES - 3)))

    lps = [_prep_layer(lp) for lp in params["layers"]]

    h, ta, tb = _init_call(
        x, coord16, params["emb_in"]["W"], params["emb_in"]["b"][None, :],
        lps[0]["w1a"], lps[0]["w1b"])

    for li, lp in enumerate(lps):
        ga, gb = _sc_gather(ta, tb, rowg, colg)
        mt = _edge_call(ga, gb, ea_pad, lp)
        aggp = _sc_scatter(mt, rows)
        if li + 1 < len(lps):
            h, ta, tb = _node_call(h, ta, aggp[0], aggp[1], lp,
                                   lps[li + 1]["w1a"], lps[li + 1]["w1b"])
        else:
            out = _tail_call(h, aggp[0], aggp[1], lp,
                             params["emb_out"]["W"],
                             params["emb_out"]["b"][None, :],
                             params["fc"]["W"], params["fc"]["b"][None, :])
    return out


# speedup vs baseline: 3.5051x; 1.1167x over previous
import functools

import jax
import jax.numpy as jnp
from jax import lax
from jax.experimental import pallas as pl
from jax.experimental.pallas import tpu as pltpu
from jax.experimental.pallas import tpu_sc as plsc

H = 64
W = 128
N_NODES = 10000
N_PAD = 10112
E_EDGES = 320000
E_PAD = 327680
NC, NS, LANES = 2, 16, 16
NW = NC * NS
EPT = E_PAD // NW
CH = 128
SCH = 256
TCH = 128
NPAIR = EPT // 1024
RPS = N_PAD // NS
BE = 2560

_MESH = plsc.VectorSubcoreMesh(core_axis_name="c", subcore_axis_name="s",
                               num_cores=NC, num_subcores=NS)


def _silu(v):
    return v * jax.nn.sigmoid(v)


def _sc_gather_body(ta_h, tb_h, rowg_h, colg_h, og_h,
                    ridx, cidx, a0, a1, b0, b1,
                    sa0, sa1, sb0, sb1, sw0, sw1):
    wid = lax.axis_index("s") * NC + lax.axis_index("c")
    base = wid * EPT
    gbase0 = wid * (EPT // 128)
    nch = EPT // CH
    for g in range(nch // 8):
        pltpu.sync_copy(rowg_h.at[pl.ds(gbase0 + g * 8, 8)], ridx.at[pl.ds(g * 8, 8)])
        pltpu.sync_copy(colg_h.at[pl.ds(gbase0 + g * 8, 8)], cidx.at[pl.ds(g * 8, 8)])

    def fire(c, a, b, sa, sb):
        pltpu.async_copy(ta_h.at[ridx.at[c]], a, sa)
        pltpu.async_copy(tb_h.at[cidx.at[c]], b, sb)

    def wait_g(a, b, sa, sb):
        pltpu.make_async_copy(ta_h.at[pl.ds(0, CH)], a, sa).wait()
        pltpu.make_async_copy(tb_h.at[pl.ds(0, CH)], b, sb).wait()

    def wait_w(a, sw):
        pltpu.make_async_copy(a, og_h.at[pl.ds(0, CH)], sw).wait()

    def comp(a, b):
        def crow(r, carry):
            for cc in range(H // LANES):
                sl = pl.ds(cc * LANES, LANES)
                a[r, sl] = a[r, sl] + b[r, sl]
            sl = pl.ds(H, LANES)
            a[r, sl] = a[r, sl] - b[r, sl]
            return carry
        lax.fori_loop(0, CH, crow, 0)

    fire(0, a0, b0, sa0, sb0)

    def pairstep(s, carry):
        c0 = 2 * s

        @pl.when(s > 0)
        def _():
            wait_w(a1, sw1)
        fire(c0 + 1, a1, b1, sa1, sb1)
        wait_g(a0, b0, sa0, sb0)
        comp(a0, b0)
        pltpu.async_copy(a0, og_h.at[pl.ds(base + c0 * CH, CH)], sw0)
        wait_g(a1, b1, sa1, sb1)
        comp(a1, b1)
        pltpu.async_copy(a1, og_h.at[pl.ds(base + (c0 + 1) * CH, CH)], sw1)
        wait_w(a0, sw0)

        @pl.when(s < nch // 2 - 1)
        def _():
            fire(c0 + 2, a0, b0, sa0, sb0)
        return carry

    lax.fori_loop(0, nch // 2, pairstep, 0)
    wait_w(a1, sw1)


def _sc_gather(ta, tb, rowg, colg):
    f = pl.kernel(
        _sc_gather_body,
        out_type=jax.ShapeDtypeStruct((E_PAD, W), jnp.float32),
        mesh=_MESH,
        scratch_types=[
            pltpu.VMEM((EPT // 128, 128), jnp.int32),
            pltpu.VMEM((EPT // 128, 128), jnp.int32),
            pltpu.VMEM((CH, W), jnp.float32),
            pltpu.VMEM((CH, W), jnp.float32),
            pltpu.VMEM((CH, W), jnp.float32),
            pltpu.VMEM((CH, W), jnp.float32),
            pltpu.SemaphoreType.DMA,
            pltpu.SemaphoreType.DMA,
            pltpu.SemaphoreType.DMA,
            pltpu.SemaphoreType.DMA,
            pltpu.SemaphoreType.DMA,
            pltpu.SemaphoreType.DMA,
        ],
    )
    return f(ta, tb, rowg, colg)


def _sc_scatter_body(mt_h, rows_h, aggp_h, idx, mbuf, aggs):
    cid = lax.axis_index("c")
    sid = lax.axis_index("s")
    wid = sid * NC + cid
    base = wid * EPT

    def zrow(r, carry):
        for c in range(W // LANES):
            mbuf[r, pl.ds(c * LANES, LANES)] = jnp.zeros((LANES,), jnp.float32)
        return carry

    lax.fori_loop(0, SCH, zrow, 0)
    ob = sid * RPS
    full = RPS // SCH
    for i in range(full):
        pltpu.sync_copy(mbuf, aggs.at[pl.ds(ob + i * SCH, SCH)])
    rem = RPS - full * SCH
    if rem:
        pltpu.sync_copy(mbuf.at[pl.ds(0, rem)],
                        aggs.at[pl.ds(ob + full * SCH, rem)])
    plsc.subcore_barrier()
    gbase0 = wid * (EPT // 128)

    def group(gi, carry):
        pltpu.sync_copy(rows_h.at[pl.ds(gbase0 + gi * 8, 8)], idx)
        for q in range(1024 // SCH):
            ebase = base + gi * 1024 + q * SCH
            pltpu.sync_copy(mt_h.at[pl.ds(ebase, SCH)], mbuf)
            for j in range(SCH // 128):
                jr = q * (SCH // 128) + j
                d = pl.ds(j * 128, 128)
                pltpu.sync_copy(mbuf.at[d], aggs.at[idx.at[jr]], add=True)
        return carry

    lax.fori_loop(0, NPAIR, group, 0)
    plsc.subcore_barrier()
    rs = pl.ds(sid * RPS, RPS)
    pltpu.sync_copy(aggs.at[rs], aggp_h.at[cid].at[rs])


def _sc_scatter(mt, rows):
    f = pl.kernel(
        _sc_scatter_body,
        out_type=jax.ShapeDtypeStruct((NC, N_PAD, W), jnp.float32),
        mesh=_MESH,
        scratch_types=[
            pltpu.VMEM((8, 128), jnp.int32),
            pltpu.VMEM((SCH, W), jnp.float32),
            pltpu.VMEM_SHARED((N_PAD, W), jnp.float32),
        ],
    )
    return f(mt, rows)


def _init_body(x_ref, c16_ref, w_ref, b_ref, w1a_ref, w1b_ref,
               h_ref, ta_ref, tb_ref):
    h = jnp.dot(x_ref[...], w_ref[...], preferred_element_type=jnp.float32) + b_ref[...]
    h_ref[...] = h
    n = h.shape[0]
    z = jnp.zeros((n, W - H - LANES), jnp.float32)
    c16 = c16_ref[...]
    ta = jnp.dot(h, w1a_ref[...], preferred_element_type=jnp.float32)
    tb = jnp.dot(h, w1b_ref[...], preferred_element_type=jnp.float32)
    ta_ref[...] = jnp.concatenate([ta, c16, z], axis=1)
    tb_ref[...] = jnp.concatenate([tb, c16, z], axis=1)


def _init_call(x, coord16, w, b, w1a, w1b):
    n = x.shape[0]
    return pl.pallas_call(
        _init_body,
        out_shape=[
            jax.ShapeDtypeStruct((n, H), jnp.float32),
            jax.ShapeDtypeStruct((n, W), jnp.float32),
            jax.ShapeDtypeStruct((n, W), jnp.float32),
        ],
    )(x, coord16, w, b, w1a, w1b)


def _edge_body(og_ref, ea_ref, wr_ref, wa_ref,
               b1_ref, w2_ref, b2_ref, wc1_ref, bc1_ref, wc2_ref, mt_ref):
    og = og_ref[...]
    cd = og[:, H:H + LANES]
    rad = jnp.sum(cd * cd, axis=1, keepdims=True)
    einp = (og[:, :H] + rad * wr_ref[...]
            + ea_ref[...] * wa_ref[...] + b1_ref[...])
    m1 = _silu(einp)
    m = _silu(jnp.dot(m1, w2_ref[...], preferred_element_type=jnp.float32)
              + b2_ref[...])
    c = _silu(jnp.dot(m, wc1_ref[...], preferred_element_type=jnp.float32)
              + bc1_ref[...])
    cw = jnp.dot(c, wc2_ref[...], preferred_element_type=jnp.float32)
    lane = lax.broadcasted_iota(jnp.int32, (BE, LANES), 1)
    trans = jnp.where(lane == 3, 1.0, cd * cw)
    z = jnp.zeros((BE, W - H - LANES), jnp.float32)
    mt_ref[...] = jnp.concatenate([m, trans, z], axis=1)


def _edge_call(og, ea, lp):
    e = og.shape[0]
    grid = (e // BE,)
    blk = lambda i: (i, 0)
    rep = lambda i: (0, 0)
    return pl.pallas_call(
        _edge_body,
        grid=grid,
        in_specs=[
            pl.BlockSpec((BE, W), blk),
            pl.BlockSpec((BE, 1), blk),
            pl.BlockSpec((1, H), rep),
            pl.BlockSpec((1, H), rep),
            pl.BlockSpec((1, H), rep),
            pl.BlockSpec((H, H), rep),
            pl.BlockSpec((1, H), rep),
            pl.BlockSpec((H, H), rep),
            pl.BlockSpec((1, H), rep),
            pl.BlockSpec((H, 1), rep),
        ],
        out_specs=pl.BlockSpec((BE, W), blk),
        out_shape=jax.ShapeDtypeStruct((e, W), jnp.float32),
    )(og, ea, lp["wr"], lp["wa"], lp["b1"], lp["w2"], lp["b2"],
      lp["wc1"], lp["bc1"], lp["wc2"])


def _node_body(h_ref, tap_ref, p0_ref, p1_ref,
               wn1a_ref, wn1b_ref, bn1_ref,
               wn2_ref, bn2_ref, w1a_ref, w1b_ref,
               h_ref_o, ta_ref, tb_ref):
    h = h_ref[...]
    n = h.shape[0]
    p = (p0_ref[...] + p1_ref[...])[:n]
    agg = p[:, :H]
    ts = p[:, H:H + LANES]
    cnt = jnp.clip(ts[:, 3:4], 1.0)
    lane = lax.broadcasted_iota(jnp.int32, ts.shape, 1)
    c16 = tap_ref[...][:, H:H + LANES] + jnp.where(lane < 3, ts / cnt, 0.0)
    u1 = _silu(jnp.dot(h, wn1a_ref[...], preferred_element_type=jnp.float32)
               + jnp.dot(agg, wn1b_ref[...], preferred_element_type=jnp.float32)
               + bn1_ref[...])
    hn = h + jnp.dot(u1, wn2_ref[...], preferred_element_type=jnp.float32) + bn2_ref[...]
    h_ref_o[...] = hn
    z = jnp.zeros((n, W - H - LANES), jnp.float32)
    ta = jnp.dot(hn, w1a_ref[...], preferred_element_type=jnp.float32)
    tb = jnp.dot(hn, w1b_ref[...], preferred_element_type=jnp.float32)
    ta_ref[...] = jnp.concatenate([ta, c16, z], axis=1)
    tb_ref[...] = jnp.concatenate([tb, c16, z], axis=1)


def _node_call(h, ta_prev, p0, p1, lp, w1a_next, w1b_next):
    n = h.shape[0]
    return pl.pallas_call(
        _node_body,
        out_shape=[
            jax.ShapeDtypeStruct((n, H), jnp.float32),
            jax.ShapeDtypeStruct((n, W), jnp.float32),
            jax.ShapeDtypeStruct((n, W), jnp.float32),
        ],
    )(h, ta_prev, p0, p1, lp["wn1a"], lp["wn1b"], lp["bn1"],
      lp["wn2"], lp["bn2"], w1a_next, w1b_next)


def _tail_body(h_ref, p0_ref, p1_ref, wn1a_ref, wn1b_ref, bn1_ref,
               wn2_ref, bn2_ref, wo_ref, bo_ref, wf_ref, bf_ref, out_ref):
    h = h_ref[...]
    n = h.shape[0]
    agg = (p0_ref[...] + p1_ref[...])[:n, :H]
    u1 = _silu(jnp.dot(h, wn1a_ref[...], preferred_element_type=jnp.float32)
               + jnp.dot(agg, wn1b_ref[...], preferred_element_type=jnp.float32)
               + bn1_ref[...])
    hn = h + jnp.dot(u1, wn2_ref[...], preferred_element_type=jnp.float32) + bn2_ref[...]
    ho = jnp.dot(hn, wo_ref[...], preferred_element_type=jnp.float32) + bo_ref[...]
    g = jnp.mean(ho, axis=0, keepdims=True)
    out_ref[...] = jnp.dot(g, wf_ref[...], preferred_element_type=jnp.float32) + bf_ref[...]


def _tail_call(h, p0, p1, lp, wo, bo, wf, bf):
    d_out = wf.shape[1]
    return pl.pallas_call(
        _tail_body,
        out_shape=jax.ShapeDtypeStruct((1, d_out), jnp.float32),
    )(h, p0, p1, lp["wn1a"], lp["wn1b"], lp["bn1"], lp["wn2"], lp["bn2"],
      wo, bo, wf, bf)


def _prep_layer(lp):
    w1 = lp["e1"]["W"]
    n1 = lp["n1"]["W"]
    return {
        "w1a": w1[:H],
        "w1b": w1[H:2 * H],
        "wr": w1[2 * H:2 * H + 1],
        "wa": w1[2 * H + 1:2 * H + 2],
        "b1": lp["e1"]["b"][None, :],
        "w2": lp["e2"]["W"],
        "b2": lp["e2"]["b"][None, :],
        "wc1": lp["c1"]["W"],
        "bc1": lp["c1"]["b"][None, :],
        "wc2": lp["c2"]["W"],
        "wn1a": n1[:H],
        "wn1b": n1[H:],
        "bn1": lp["n1"]["b"][None, :],
        "wn2": lp["n2"]["W"],
        "bn2": lp["n2"]["b"][None, :],
    }


def kernel(x, edge_index, coord, edge_attr, params):
    row, col = edge_index[0], edge_index[1]
    pe = E_PAD - E_EDGES
    rowg = jnp.pad(row, (0, pe)).reshape(E_PAD // 128, 128)
    colg = jnp.pad(col, (0, pe)).reshape(E_PAD // 128, 128)
    rows = jnp.pad(row, (0, pe), constant_values=N_NODES).reshape(E_PAD // 128, 128)
    ea_pad = jnp.pad(edge_attr, ((0, pe), (0, 0)))
    coord16 = jnp.pad(coord, ((0, 0), (0, LANES - 3)))

    lps = [_prep_layer(lp) for lp in params["layers"]]

    h, ta, tb = _init_call(
        x, coord16, params["emb_in"]["W"], params["emb_in"]["b"][None, :],
        lps[0]["w1a"], lps[0]["w1b"])

    for li, lp in enumerate(lps):
        og = _sc_gather(ta, tb, rowg, colg)
        mt = _edge_call(og, ea_pad, lp)
        aggp = _sc_scatter(mt, rows)
        if li + 1 < len(lps):
            h, ta, tb = _node_call(h, ta, aggp[0], aggp[1], lp,
                                   lps[li + 1]["w1a"], lps[li + 1]["w1b"])
        else:
            out = _tail_call(h, aggp[0], aggp[1], lp,
                             params["emb_out"]["W"],
                             params["emb_out"]["b"][None, :],
                             params["fc"]["W"], params["fc"]["b"][None, :])
    return out
